# baseline (speedup 1.0000x reference)
import numpy as np
import jax
import jax.numpy as jnp
from jax import lax
from jax.experimental import pallas as pl
from jax.experimental.pallas import tpu as pltpu
from jax.experimental.pallas import tpu_sc as plsc

BATCH = 1024
NPART = 16384
NQ = 20
L = 16
NC, NS = 2, 16
NW = NC * NS
ROWS_PER_TILE = BATCH // NW
NVEC = NPART // L
NRP = 48
DUMMY = 47
QPAD = 32
UNROLL = 16
CUN = 4
SIGN = np.int32(-2147483648)
MASK31 = np.int32(2147483647)


def _rank_consts():
    qs = np.array([
        1028443341, 1036478745, 1041511909, 1044690750, 1047869591,
        1049812216, 1051401637, 1052991057, 1054580478, 1056169898,
        1057361963, 1058156674, 1058951384, 1059746094, 1060540804,
        1061335514, 1062130225, 1062924935, 1063719645, 1064514355,
    ], dtype=np.uint32).view(np.float32)
    pos = (qs * np.float32(NPART - 1)).astype(np.float32)
    lo = np.clip(np.floor(pos), 0, NPART - 1).astype(np.int64)
    hi = np.clip(np.ceil(pos), 0, NPART - 1).astype(np.int64)
    w = (pos - lo.astype(np.float32)).astype(np.float32)
    ranks = np.empty(NRP, dtype=np.int32)
    ranks[0:2 * NQ:2] = lo
    ranks[1:2 * NQ:2] = hi
    ranks[2 * NQ:] = hi[-1]
    return ranks, w


_RANKS, _HIW = _rank_consts()


def _to_sortable(xf):
    bits = lax.bitcast_convert_type(xf, jnp.int32)
    m = lax.shift_right_arithmetic(bits, 31)
    return lax.bitwise_xor(bits, lax.bitwise_or(SIGN, lax.bitwise_and(MASK31, m)))


def _from_sortable(u):
    m = lax.shift_right_arithmetic(u, 31)
    notm = lax.bitwise_xor(m, np.int32(-1))
    bits = lax.bitwise_xor(u, lax.bitwise_or(SIGN, lax.bitwise_and(MASK31, notm)))
    return lax.bitcast_convert_type(bits, jnp.float32)


def _srl(x, n):
    return lax.shift_right_logical(x, np.int32(n))


def _sc_quantile_body(x_hbm, rk_hbm, hw_hbm, qout_hbm, data, hist1, table2,
                      table3, table4, hist2, wa, cb, stage, vstage, qrow,
                      rk_v, hw_v, dsem):
    wid = lax.axis_index("s") * NC + lax.axis_index("c")
    lane = lax.iota(jnp.int32, L)
    zeros = lane * 0
    ones = zeros + 1
    dummy = zeros + DUMMY
    fzeros = zeros.astype(jnp.float32)
    tsel = lax.shift_left(lax.bitwise_and(lane, np.int32(3)), np.int32(11))
    pltpu.sync_copy(rk_hbm, rk_v)
    pltpu.sync_copy(hw_hbm, hw_v)
    kvecs = [rk_v[pl.ds(i * L, L)] for i in range(3)]
    hiw0 = hw_v[pl.ds(0, L)]
    hiw1 = hw_v[pl.ds(L, L)]

    def init1(i, _):
        for j in range(4):
            table2[pl.ds((i * 4 + j) * L, L)] = dummy
        for j in range(16):
            hist1[pl.ds((i * 16 + j) * L, L)] = zeros
        return 0
    lax.fori_loop(0, 2048 // L // 4, init1, 0)

    def init2(i, _):
        for j in range(4):
            hist2[pl.ds((i * 4 + j) * L, L)] = zeros
            table3[pl.ds((i * 4 + j) * L, L)] = dummy
            table4[pl.ds((i * 4 + j) * L, L)] = dummy
        return 0
    lax.fori_loop(0, (NRP * 128) // L // 4, init2, 0)

    def rank_prep(table, pkeys):
        stage[pl.ds(1, L)] = pkeys[0]
        stage[pl.ds(1 + L, L)] = pkeys[1]
        stage[pl.ds(1 + 2 * L, L)] = pkeys[2]
        sh = [stage[pl.ds(i * L, L)] for i in range(3)]
        f0 = jnp.logical_or(pkeys[0] != sh[0], lane == 0)
        f1 = pkeys[1] != sh[1]
        f2 = pkeys[2] != sh[2]
        fi = [f0.astype(jnp.int32), f1.astype(jnp.int32), f2.astype(jnp.int32)]
        n0 = jnp.sum(fi[0])
        n1 = jnp.sum(fi[1])
        slots = [plsc.cumsum(fi[0]) - 1,
                 plsc.cumsum(fi[1]) + (n0 - 1),
                 plsc.cumsum(fi[2]) + (n0 + n1 - 1)]
        plsc.store_scatter(table, [pkeys[0]], slots[0], mask=f0)
        plsc.store_scatter(table, [pkeys[1]], slots[1], mask=f1)
        plsc.store_scatter(table, [pkeys[2]], slots[2], mask=f2)
        return slots

    def hist_scan(slots, kks):
        SCU = 8

        def sc(b0, carry):
            acc, bins, bef = carry
            hs = []
            for j in range(SCU):
                base = (b0 * SCU + j) * NRP
                hs.append([plsc.load_gather(hist2, [slots[g] + base])
                           for g in range(3)])
            for j in range(SCU):
                acc2, bins2, bef2 = [], [], []
                for g in range(3):
                    a = acc[g] + hs[j][g]
                    m = a <= kks[g]
                    acc2.append(a)
                    bins2.append(bins[g] + m.astype(jnp.int32))
                    bef2.append(jnp.where(m, a, bef[g]))
                acc, bins, bef = tuple(acc2), tuple(bins2), tuple(bef2)
            for j in range(SCU):
                base = (b0 * SCU + j) * NRP
                hist2[pl.ds(base, L)] = zeros
                hist2[pl.ds(base + L, L)] = zeros
                hist2[pl.ds(base + 2 * L, L)] = zeros
            return (acc, bins, bef)
        z3 = (zeros, zeros, zeros)
        _, bins, bef = lax.fori_loop(0, 128 // SCU, sc, (z3, z3, z3))
        return bins, bef

    row0 = wid * ROWS_PER_TILE
    pltpu.make_async_copy(x_hbm.at[row0], data.at[0], dsem).start()

    def row_body(i, _):
        buf = lax.rem(i, 2)
        pltpu.make_async_copy(x_hbm.at[row0 + i], data.at[buf], dsem).wait()

        @pl.when(i + 1 < ROWS_PER_TILE)
        def _prefetch():
            pltpu.make_async_copy(x_hbm.at[row0 + i + 1], data.at[1 - buf],
                                  dsem).start()

        def load_u(v):
            return _to_sortable(data[buf, pl.ds(v * L, L)])

        def p1(v, _):
            us = [load_u(v * UNROLL + j) for j in range(UNROLL)]
            keys = [_srl(u, 21) + tsel for u in us]
            for j in range(UNROLL):
                plsc.addupdate_scatter(hist1, [keys[j]], ones)
            for j in range(UNROLL):
                data[buf, pl.ds((v * UNROLL + j) * L, L)] = (
                    lax.bitcast_convert_type(us[j], jnp.float32))
            return 0
        lax.fori_loop(0, NVEC // UNROLL, p1, 0)

        def cs(b, carry):
            hs = [hist1[pl.ds(t * 2048 + b * L, L)] for t in range(4)]
            h = (hs[0] + hs[1]) + (hs[2] + hs[3])
            c = plsc.cumsum(h) + carry
            hist1[pl.ds(b * L, L)] = c
            for t in range(1, 4):
                hist1[pl.ds(t * 2048 + b * L, L)] = zeros
            return c[L - 1]
        lax.fori_loop(0, 2048 // L, cs, np.int32(0))

        def bsearch(kvec):
            b = zeros
            for j in (1024, 512, 256, 128, 64, 32, 16, 8, 4, 2, 1):
                val = plsc.load_gather(hist1, [b + (j - 1)])
                b = jnp.where(val <= kvec, b + j, b)
            return b
        b1 = [bsearch(kvecs[g]) for g in range(3)]
        kks = []
        for g in range(3):
            prev = plsc.load_gather(hist1, [jnp.maximum(b1[g] - 1, 0)])
            bef = jnp.where(b1[g] > 0, prev, 0)
            kks.append(kvecs[g] - bef)
        kks = tuple(kks)

        def clr(b, _):
            for j in range(4):
                hist1[pl.ds((b * 4 + j) * L, L)] = zeros
            return 0
        lax.fori_loop(0, 2048 // L // 4, clr, 0)

        pk2 = tuple(b1)
        s2 = rank_prep(table2, pk2)

        def dp2(v, _):
            us = [lax.bitcast_convert_type(
                data[buf, pl.ds((v * UNROLL + j) * L, L)], jnp.int32)
                for j in range(UNROLL)]
            ss = [plsc.load_gather(table2, [_srl(u, 21)]) for u in us]
            idxs = [lax.bitwise_and(_srl(u, 14), np.int32(127)) * NRP + s
                    for s, u in zip(ss, us)]
            ws = [lax.bitwise_or(lax.shift_left(s, 21),
                                 lax.bitwise_and(u, np.int32(0x1FFFFF)))
                  for s, u in zip(ss, us)]
            for j in range(UNROLL):
                plsc.addupdate_scatter(hist2, [idxs[j]], ones)
            for j in range(UNROLL):
                wa[pl.ds((v * UNROLL + j) * L, L)] = ws[j]
            return 0
        lax.fori_loop(0, NVEC // UNROLL, dp2, 0)
        b2, bef = hist_scan(s2, kks)
        kks = tuple(kks[g] - bef[g] for g in range(3))

        pk3 = tuple(s2[g] * 128 + b2[g] for g in range(3))
        s3 = rank_prep(table3, pk3)

        def dp3(v, off):
            ws = [wa[pl.ds((v * UNROLL + j) * L, L)] for j in range(UNROLL)]
            sb = [plsc.load_gather(
                table3,
                [_srl(w, 21) * 128 + lax.bitwise_and(_srl(w, 14), np.int32(127))])
                for w in ws]
            idxs = [lax.bitwise_and(_srl(w, 7), np.int32(127)) * NRP + s
                    for s, w in zip(sb, ws)]
            cm = [s != DUMMY for s in sb]
            cnt = [jnp.sum(m.astype(jnp.int32)) for m in cm]
            for j in range(UNROLL):
                plsc.addupdate_scatter(hist2, [idxs[j]], ones)
            for j in range(UNROLL):
                plsc.store_compressed(cb.at[pl.ds(off, L)], ws[j], mask=cm[j])
                off = off + cnt[j]
            return off
        cb_n = lax.fori_loop(0, NVEC // UNROLL, dp3, np.int32(0))
        b3, bef = hist_scan(s3, kks)
        kks = tuple(kks[g] - bef[g] for g in range(3))

        pk4 = tuple(s3[g] * 128 + b3[g] for g in range(3))
        s4 = rank_prep(table4, pk4)

        def dp4(v, _):
            for j in range(CUN):
                base = (v * CUN + j) * L
                w = cb[pl.ds(base, L)]
                k2 = lax.bitwise_and(_srl(w, 14), np.int32(127))
                s2e = jnp.minimum(_srl(w, 21), DUMMY)
                sb = plsc.load_gather(table3, [s2e * 128 + k2])
                k3 = lax.bitwise_and(_srl(w, 7), np.int32(127))
                sc_ = plsc.load_gather(table4, [sb * 128 + k3])
                valid = (lane + base) < cb_n
                key = lax.bitwise_and(w, np.int32(127))
                plsc.addupdate_scatter(hist2, [key * NRP + sc_], ones,
                                       mask=valid)
            return 0
        n4 = _srl(cb_n + (CUN * L - 1), 6)
        lax.fori_loop(0, n4, dp4, 0)
        b4, bef = hist_scan(s4, kks)

        for g in range(3):
            plsc.store_scatter(table2, [pk2[g]], dummy)
            plsc.store_scatter(table3, [pk3[g]], dummy)
            plsc.store_scatter(table4, [pk4[g]], dummy)

        for g in range(3):
            u = lax.bitwise_or(
                lax.bitwise_or(lax.shift_left(b1[g], 21), lax.shift_left(b2[g], 14)),
                lax.bitwise_or(lax.shift_left(b3[g], 7), b4[g]))
            vstage[pl.ds(g * L, L)] = _from_sortable(u)
        vstage[pl.ds(3 * L, L)] = fzeros

        qlo0 = plsc.load_gather(vstage, [lane * 2])
        qhi0 = plsc.load_gather(vstage, [lane * 2 + 1])
        qv0 = qlo0 * (1.0 - hiw0) + qhi0 * hiw0
        qlo1 = plsc.load_gather(vstage, [lane * 2 + 2 * L])
        qhi1 = plsc.load_gather(vstage, [lane * 2 + 2 * L + 1])
        qv1 = qlo1 * (1.0 - hiw1) + qhi1 * hiw1
        qv1 = jnp.where(lane < (NQ - L), qv1, 0.0)
        qrow[i, pl.ds(0, L)] = qv0
        qrow[i, pl.ds(L, L)] = qv1
        return 0

    lax.fori_loop(0, ROWS_PER_TILE, row_body, 0)
    pltpu.sync_copy(qrow, qout_hbm.at[pl.ds(wid * ROWS_PER_TILE, ROWS_PER_TILE)])


def _sc_quantile(x):
    mesh = plsc.VectorSubcoreMesh(core_axis_name="c", subcore_axis_name="s",
                                  num_cores=NC, num_subcores=NS)
    f = pl.kernel(
        _sc_quantile_body,
        out_type=jax.ShapeDtypeStruct((BATCH, QPAD), jnp.float32),
        mesh=mesh,
        compiler_params=pltpu.CompilerParams(needs_layout_passes=False),
        scratch_types=[
            pltpu.VMEM((2, NPART), jnp.float32),
            pltpu.VMEM((4 * 2048,), jnp.int32),
            pltpu.VMEM((2048,), jnp.int32),
            pltpu.VMEM((NRP * 128,), jnp.int32),
            pltpu.VMEM((NRP * 128,), jnp.int32),
            pltpu.VMEM((NRP * 128,), jnp.int32),
            pltpu.VMEM((NPART,), jnp.int32),
            pltpu.VMEM((NPART + CUN * L,), jnp.int32),
            pltpu.VMEM((64,), jnp.int32),
            pltpu.VMEM((4 * L,), jnp.float32),
            pltpu.VMEM((ROWS_PER_TILE, QPAD), jnp.float32),
            pltpu.VMEM((NRP,), jnp.int32),
            pltpu.VMEM((QPAD,), jnp.float32),
            pltpu.SemaphoreType.DMA,
        ],
    )
    hw = np.pad(_HIW, (0, QPAD - NQ)).astype(np.float32)
    return f(x, jnp.asarray(_RANKS), jnp.asarray(hw))


def _mlp_body(q_ref, w1t_ref, b1_ref, w2t_ref, b2_ref, o_ref):
    q = q_ref[...]
    h = jnp.maximum(
        jnp.dot(q, w1t_ref[...], preferred_element_type=jnp.float32) + b1_ref[...],
        0.0)
    o_ref[...] = (jnp.dot(h, w2t_ref[...], preferred_element_type=jnp.float32)
                  + b2_ref[...])


def kernel(x, W1, b1, W2, b2):
    qpad = _sc_quantile(x)
    w1t = jnp.pad(W1.T, ((0, QPAD - NQ), (0, 0)))
    w2t = W2.T
    z = pl.pallas_call(
        _mlp_body,
        out_shape=jax.ShapeDtypeStruct((BATCH, W2.shape[0]), jnp.float32),
    )(qpad, w1t, b1.reshape(1, -1), w2t, b2.reshape(1, -1))
    return z


# speedup vs baseline: 1.0497x; 1.0497x over previous
import numpy as np
import jax
import jax.numpy as jnp
from jax import lax
from jax.experimental import pallas as pl
from jax.experimental.pallas import tpu as pltpu
from jax.experimental.pallas import tpu_sc as plsc

BATCH = 1024
NPART = 16384
NQ = 20
L = 16
NC, NS = 2, 16
NW = NC * NS
ROWS_PER_TILE = BATCH // NW
NVEC = NPART // L
NRP = 48
DUMMY = 47
QPAD = 32
UNROLL = 16
CUN = 4
SIGN = np.int32(-2147483648)
MASK31 = np.int32(2147483647)


def _rank_consts():
    qs = np.array([
        1028443341, 1036478745, 1041511909, 1044690750, 1047869591,
        1049812216, 1051401637, 1052991057, 1054580478, 1056169898,
        1057361963, 1058156674, 1058951384, 1059746094, 1060540804,
        1061335514, 1062130225, 1062924935, 1063719645, 1064514355,
    ], dtype=np.uint32).view(np.float32)
    pos = (qs * np.float32(NPART - 1)).astype(np.float32)
    lo = np.clip(np.floor(pos), 0, NPART - 1).astype(np.int64)
    hi = np.clip(np.ceil(pos), 0, NPART - 1).astype(np.int64)
    w = (pos - lo.astype(np.float32)).astype(np.float32)
    ranks = np.empty(NRP, dtype=np.int32)
    ranks[0:2 * NQ:2] = lo
    ranks[1:2 * NQ:2] = hi
    ranks[2 * NQ:] = hi[-1]
    return ranks, w


_RANKS, _HIW = _rank_consts()


def _to_sortable(xf):
    bits = lax.bitcast_convert_type(xf, jnp.int32)
    m = lax.shift_right_arithmetic(bits, 31)
    return lax.bitwise_xor(bits, lax.bitwise_or(SIGN, lax.bitwise_and(MASK31, m)))


def _from_sortable(u):
    m = lax.shift_right_arithmetic(u, 31)
    notm = lax.bitwise_xor(m, np.int32(-1))
    bits = lax.bitwise_xor(u, lax.bitwise_or(SIGN, lax.bitwise_and(MASK31, notm)))
    return lax.bitcast_convert_type(bits, jnp.float32)


def _srl(x, n):
    return lax.shift_right_logical(x, np.int32(n))


def _sc_quantile_body(x_hbm, rk_hbm, hw_hbm, qout_hbm, data, hist1, table2,
                      table3, table4, hist2, wa, cb, stage, vstage, qrow,
                      rk_v, hw_v, dsem):
    wid = lax.axis_index("s") * NC + lax.axis_index("c")
    lane = lax.iota(jnp.int32, L)
    zeros = lane * 0
    ones = zeros + 1
    dummy = zeros + DUMMY
    fzeros = zeros.astype(jnp.float32)
    pltpu.sync_copy(rk_hbm, rk_v)
    pltpu.sync_copy(hw_hbm, hw_v)
    kvecs = [rk_v[pl.ds(i * L, L)] for i in range(3)]
    hiw0 = hw_v[pl.ds(0, L)]
    hiw1 = hw_v[pl.ds(L, L)]

    def init1(i, _):
        for j in range(4):
            hist1[pl.ds((i * 4 + j) * L, L)] = zeros
            table2[pl.ds((i * 4 + j) * L, L)] = dummy
        return 0
    lax.fori_loop(0, 2048 // L // 4, init1, 0)

    def init2(i, _):
        for j in range(4):
            hist2[pl.ds((i * 4 + j) * L, L)] = zeros
            table3[pl.ds((i * 4 + j) * L, L)] = dummy
            table4[pl.ds((i * 4 + j) * L, L)] = dummy
        return 0
    lax.fori_loop(0, (NRP * 128) // L // 4, init2, 0)

    def rank_prep(table, pkeys):
        stage[pl.ds(1, L)] = pkeys[0]
        stage[pl.ds(1 + L, L)] = pkeys[1]
        stage[pl.ds(1 + 2 * L, L)] = pkeys[2]
        sh = [stage[pl.ds(i * L, L)] for i in range(3)]
        f0 = jnp.logical_or(pkeys[0] != sh[0], lane == 0)
        f1 = pkeys[1] != sh[1]
        f2 = pkeys[2] != sh[2]
        fi = [f0.astype(jnp.int32), f1.astype(jnp.int32), f2.astype(jnp.int32)]
        n0 = jnp.sum(fi[0])
        n1 = jnp.sum(fi[1])
        slots = [plsc.cumsum(fi[0]) - 1,
                 plsc.cumsum(fi[1]) + (n0 - 1),
                 plsc.cumsum(fi[2]) + (n0 + n1 - 1)]
        plsc.store_scatter(table, [pkeys[0]], slots[0], mask=f0)
        plsc.store_scatter(table, [pkeys[1]], slots[1], mask=f1)
        plsc.store_scatter(table, [pkeys[2]], slots[2], mask=f2)
        return slots

    def hist_scan(slots, kks):
        SCU = 8

        def sc(b0, carry):
            acc, bins, bef = carry
            hs = []
            for j in range(SCU):
                base = (b0 * SCU + j) * NRP
                hs.append([plsc.load_gather(hist2, [slots[g] + base])
                           for g in range(3)])
            for j in range(SCU):
                acc2, bins2, bef2 = [], [], []
                for g in range(3):
                    a = acc[g] + hs[j][g]
                    m = a <= kks[g]
                    acc2.append(a)
                    bins2.append(bins[g] + m.astype(jnp.int32))
                    bef2.append(jnp.where(m, a, bef[g]))
                acc, bins, bef = tuple(acc2), tuple(bins2), tuple(bef2)
            for j in range(SCU):
                base = (b0 * SCU + j) * NRP
                hist2[pl.ds(base, L)] = zeros
                hist2[pl.ds(base + L, L)] = zeros
                hist2[pl.ds(base + 2 * L, L)] = zeros
            return (acc, bins, bef)
        z3 = (zeros, zeros, zeros)
        _, bins, bef = lax.fori_loop(0, 128 // SCU, sc, (z3, z3, z3))
        return bins, bef

    row0 = wid * ROWS_PER_TILE
    pltpu.make_async_copy(x_hbm.at[row0], data.at[0], dsem).start()

    def row_body(i, _):
        buf = lax.rem(i, 2)
        pltpu.make_async_copy(x_hbm.at[row0 + i], data.at[buf], dsem).wait()

        @pl.when(i + 1 < ROWS_PER_TILE)
        def _prefetch():
            pltpu.make_async_copy(x_hbm.at[row0 + i + 1], data.at[1 - buf],
                                  dsem).start()

        def load_u(v):
            return _to_sortable(data[buf, pl.ds(v * L, L)])

        def p1(v, _):
            us = [load_u(v * UNROLL + j) for j in range(UNROLL)]
            keys = [_srl(u, 21) for u in us]
            for j in range(UNROLL):
                plsc.addupdate_scatter(hist1, [keys[j]], ones)
            for j in range(UNROLL):
                data[buf, pl.ds((v * UNROLL + j) * L, L)] = (
                    lax.bitcast_convert_type(us[j], jnp.float32))
            return 0
        lax.fori_loop(0, NVEC // UNROLL, p1, 0)

        def cs(b, carry):
            h = hist1[pl.ds(b * L, L)]
            c = plsc.cumsum(h) + carry
            hist1[pl.ds(b * L, L)] = c
            return c[L - 1]
        lax.fori_loop(0, 2048 // L, cs, np.int32(0))

        def bsearch(kvec):
            b = zeros
            for j in (1024, 512, 256, 128, 64, 32, 16, 8, 4, 2, 1):
                val = plsc.load_gather(hist1, [b + (j - 1)])
                b = jnp.where(val <= kvec, b + j, b)
            return b
        b1 = [bsearch(kvecs[g]) for g in range(3)]
        kks = []
        for g in range(3):
            prev = plsc.load_gather(hist1, [jnp.maximum(b1[g] - 1, 0)])
            bef = jnp.where(b1[g] > 0, prev, 0)
            kks.append(kvecs[g] - bef)
        kks = tuple(kks)

        def clr(b, _):
            for j in range(4):
                hist1[pl.ds((b * 4 + j) * L, L)] = zeros
            return 0
        lax.fori_loop(0, 2048 // L // 4, clr, 0)

        pk2 = tuple(b1)
        s2 = rank_prep(table2, pk2)

        def dp2(v, _):
            us = [lax.bitcast_convert_type(
                data[buf, pl.ds((v * UNROLL + j) * L, L)], jnp.int32)
                for j in range(UNROLL)]
            ss = [plsc.load_gather(table2, [_srl(u, 21)]) for u in us]
            idxs = [lax.bitwise_and(_srl(u, 14), np.int32(127)) * NRP + s
                    for s, u in zip(ss, us)]
            ws = [lax.bitwise_or(lax.shift_left(s, 21),
                                 lax.bitwise_and(u, np.int32(0x1FFFFF)))
                  for s, u in zip(ss, us)]
            for j in range(UNROLL):
                plsc.addupdate_scatter(hist2, [idxs[j]], ones)
            for j in range(UNROLL):
                wa[pl.ds((v * UNROLL + j) * L, L)] = ws[j]
            return 0
        lax.fori_loop(0, NVEC // UNROLL, dp2, 0)
        b2, bef = hist_scan(s2, kks)
        kks = tuple(kks[g] - bef[g] for g in range(3))

        pk3 = tuple(s2[g] * 128 + b2[g] for g in range(3))
        s3 = rank_prep(table3, pk3)

        def dp3(v, off):
            ws = [wa[pl.ds((v * UNROLL + j) * L, L)] for j in range(UNROLL)]
            sb = [plsc.load_gather(
                table3,
                [_srl(w, 21) * 128 + lax.bitwise_and(_srl(w, 14), np.int32(127))])
                for w in ws]
            idxs = [lax.bitwise_and(_srl(w, 7), np.int32(127)) * NRP + s
                    for s, w in zip(sb, ws)]
            cm = [s != DUMMY for s in sb]
            cnt = [jnp.sum(m.astype(jnp.int32)) for m in cm]
            for j in range(UNROLL):
                plsc.addupdate_scatter(hist2, [idxs[j]], ones)
            for j in range(UNROLL):
                plsc.store_compressed(cb.at[pl.ds(off, L)], ws[j], mask=cm[j])
                off = off + cnt[j]
            return off
        cb_n = lax.fori_loop(0, NVEC // UNROLL, dp3, np.int32(0))
        b3, bef = hist_scan(s3, kks)
        kks = tuple(kks[g] - bef[g] for g in range(3))

        pk4 = tuple(s3[g] * 128 + b3[g] for g in range(3))
        s4 = rank_prep(table4, pk4)

        def dp4(v, _):
            for j in range(CUN):
                base = (v * CUN + j) * L
                w = cb[pl.ds(base, L)]
                k2 = lax.bitwise_and(_srl(w, 14), np.int32(127))
                s2e = jnp.minimum(_srl(w, 21), DUMMY)
                sb = plsc.load_gather(table3, [s2e * 128 + k2])
                k3 = lax.bitwise_and(_srl(w, 7), np.int32(127))
                sc_ = plsc.load_gather(table4, [sb * 128 + k3])
                valid = (lane + base) < cb_n
                key = lax.bitwise_and(w, np.int32(127))
                plsc.addupdate_scatter(hist2, [key * NRP + sc_], ones,
                                       mask=valid)
            return 0
        n4 = _srl(cb_n + (CUN * L - 1), 6)
        lax.fori_loop(0, n4, dp4, 0)
        b4, bef = hist_scan(s4, kks)

        for g in range(3):
            plsc.store_scatter(table2, [pk2[g]], dummy)
            plsc.store_scatter(table3, [pk3[g]], dummy)
            plsc.store_scatter(table4, [pk4[g]], dummy)

        for g in range(3):
            u = lax.bitwise_or(
                lax.bitwise_or(lax.shift_left(b1[g], 21), lax.shift_left(b2[g], 14)),
                lax.bitwise_or(lax.shift_left(b3[g], 7), b4[g]))
            vstage[pl.ds(g * L, L)] = _from_sortable(u)
        vstage[pl.ds(3 * L, L)] = fzeros

        qlo0 = plsc.load_gather(vstage, [lane * 2])
        qhi0 = plsc.load_gather(vstage, [lane * 2 + 1])
        qv0 = qlo0 * (1.0 - hiw0) + qhi0 * hiw0
        qlo1 = plsc.load_gather(vstage, [lane * 2 + 2 * L])
        qhi1 = plsc.load_gather(vstage, [lane * 2 + 2 * L + 1])
        qv1 = qlo1 * (1.0 - hiw1) + qhi1 * hiw1
        qv1 = jnp.where(lane < (NQ - L), qv1, 0.0)
        qrow[i, pl.ds(0, L)] = qv0
        qrow[i, pl.ds(L, L)] = qv1
        return 0

    lax.fori_loop(0, ROWS_PER_TILE, row_body, 0)
    pltpu.sync_copy(qrow, qout_hbm.at[pl.ds(wid * ROWS_PER_TILE, ROWS_PER_TILE)])


def _sc_quantile(x):
    mesh = plsc.VectorSubcoreMesh(core_axis_name="c", subcore_axis_name="s",
                                  num_cores=NC, num_subcores=NS)
    f = pl.kernel(
        _sc_quantile_body,
        out_type=jax.ShapeDtypeStruct((BATCH, QPAD), jnp.float32),
        mesh=mesh,
        compiler_params=pltpu.CompilerParams(needs_layout_passes=False),
        scratch_types=[
            pltpu.VMEM((2, NPART), jnp.float32),
            pltpu.VMEM((2048,), jnp.int32),
            pltpu.VMEM((2048,), jnp.int32),
            pltpu.VMEM((NRP * 128,), jnp.int32),
            pltpu.VMEM((NRP * 128,), jnp.int32),
            pltpu.VMEM((NRP * 128,), jnp.int32),
            pltpu.VMEM((NPART,), jnp.int32),
            pltpu.VMEM((NPART + CUN * L,), jnp.int32),
            pltpu.VMEM((64,), jnp.int32),
            pltpu.VMEM((4 * L,), jnp.float32),
            pltpu.VMEM((ROWS_PER_TILE, QPAD), jnp.float32),
            pltpu.VMEM((NRP,), jnp.int32),
            pltpu.VMEM((QPAD,), jnp.float32),
            pltpu.SemaphoreType.DMA,
        ],
    )
    hw = np.pad(_HIW, (0, QPAD - NQ)).astype(np.float32)
    return f(x, jnp.asarray(_RANKS), jnp.asarray(hw))


def _mlp_body(q_ref, w1t_ref, b1_ref, w2t_ref, b2_ref, o_ref):
    q = q_ref[...]
    h = jnp.maximum(
        jnp.dot(q, w1t_ref[...], preferred_element_type=jnp.float32) + b1_ref[...],
        0.0)
    o_ref[...] = (jnp.dot(h, w2t_ref[...], preferred_element_type=jnp.float32)
                  + b2_ref[...])


def kernel(x, W1, b1, W2, b2):
    qpad = _sc_quantile(x)
    w1t = jnp.pad(W1.T, ((0, QPAD - NQ), (0, 0)))
    w2t = W2.T
    z = pl.pallas_call(
        _mlp_body,
        out_shape=jax.ShapeDtypeStruct((BATCH, W2.shape[0]), jnp.float32),
    )(qpad, w1t, b1.reshape(1, -1), w2t, b2.reshape(1, -1))
    return z
